# baseline (speedup 1.0000x reference)
import functools

import jax
import jax.numpy as jnp
from jax import lax
from jax.experimental import pallas as pl
from jax.experimental.pallas import tpu as pltpu
from jax.experimental.pallas import tpu_sc as plsc

_V = 1000000
_D = 64
_B = 16384
_NEG = 10

_NC = 2
_NS = 16
_NW = _NC * _NS
_PER_W = _B // _NW
_CHUNK = 64
_STEPS = _PER_W // _CHUNK
_NIDX = _NEG + 2


def _sc_gather_dots(w_in, w_ctx, iw, cw, neg_t, img, s4):
  f32 = jnp.float32
  mesh = plsc.VectorSubcoreMesh(core_axis_name="c", subcore_axis_name="s")

  @functools.partial(
      pl.kernel,
      out_type=(
          jax.ShapeDtypeStruct((_B,), f32),
          jax.ShapeDtypeStruct((_B * _NEG,), f32),
          jax.ShapeDtypeStruct((_B,), f32),
          jax.ShapeDtypeStruct((_B,), f32),
          jax.ShapeDtypeStruct((_B,), f32),
          jax.ShapeDtypeStruct((_B,), f32),
      ),
      mesh=mesh,
      compiler_params=pltpu.CompilerParams(
          needs_layout_passes=False, use_tc_tiling_on_sc=False),
      scratch_types=[
          pltpu.VMEM((_NIDX, _PER_W), jnp.int32),
          pltpu.VMEM((_CHUNK, _D), f32),
          pltpu.VMEM((_CHUNK, _D), f32),
          pltpu.VMEM((_NEG, _CHUNK, _D), f32),
          pltpu.VMEM((_CHUNK, _D), f32),
          pltpu.VMEM((_CHUNK, _D), f32),
          pltpu.VMEM((_CHUNK,), f32),
          pltpu.VMEM((_CHUNK * _NEG,), f32),
          pltpu.VMEM((_CHUNK,), f32),
          pltpu.VMEM((_CHUNK,), f32),
          pltpu.VMEM((_CHUNK,), f32),
          pltpu.VMEM((_CHUNK,), f32),
          pltpu.VMEM((_CHUNK,), f32),
          pltpu.VMEM((_CHUNK,), f32),
          pltpu.VMEM((_CHUNK,), f32),
          pltpu.SemaphoreType.DMA,
      ],
  )
  def sc(w_in_hbm, w_ctx_hbm, iw_hbm, cw_hbm, negt_hbm, img_hbm, s4_hbm,
         pos_hbm, negs_hbm, dimg_hbm, ds4_hbm, denp_hbm, denn_hbm,
         idx_v, rows_in, rows_ctx, rows_neg, img_v, s4_v,
         pos_v, negs_v, dimg_v, ds4_v, nin_v, nimg_v, ns4_v, denp_v, denn_v,
         sem):
    wid = lax.axis_index("s") * _NC + lax.axis_index("c")
    wbase = wid * _PER_W
    pltpu.sync_copy(iw_hbm.at[pl.ds(wbase, _PER_W)], idx_v.at[0])
    pltpu.sync_copy(cw_hbm.at[pl.ds(wbase, _PER_W)], idx_v.at[1])
    for k in range(_NEG):
      pltpu.sync_copy(negt_hbm.at[k, pl.ds(wbase, _PER_W)], idx_v.at[2 + k])

    for s in range(_STEPS):
      off = s * _CHUNK
      gbase = wbase + off
      cps = [
          pltpu.async_copy(
              w_in_hbm.at[idx_v.at[0, pl.ds(off, _CHUNK)]], rows_in, sem),
          pltpu.async_copy(
              w_ctx_hbm.at[idx_v.at[1, pl.ds(off, _CHUNK)]], rows_ctx, sem),
      ]
      for k in range(_NEG):
        cps.append(pltpu.async_copy(
            w_ctx_hbm.at[idx_v.at[2 + k, pl.ds(off, _CHUNK)]],
            rows_neg.at[k], sem))
      pltpu.sync_copy(img_hbm.at[pl.ds(gbase, _CHUNK), :], img_v)
      pltpu.sync_copy(s4_hbm.at[pl.ds(gbase, _CHUNK), :], s4_v)
      for cp in cps:
        cp.wait()

      lane = lax.iota(jnp.int32, 16)
      m15 = lane == 15

      def hstore(ref, offvec, acc):
        plsc.store_scatter(ref, [offvec], plsc.cumsum(acc), mask=m15)

      def dot_body(e, carry):
        eidx = jnp.full((16,), e, dtype=jnp.int32)
        a = [rows_in[e, pl.ds(16 * j, 16)] for j in range(4)]
        c = [rows_ctx[e, pl.ds(16 * j, 16)] for j in range(4)]
        acc = a[0] * c[0]
        for j in range(1, 4):
          acc += a[j] * c[j]
        hstore(pos_v, eidx, acc)
        for k in range(_NEG):
          acc = a[0] * rows_neg[k, e, pl.ds(0, 16)]
          for j in range(1, 4):
            acc += a[j] * rows_neg[k, e, pl.ds(16 * j, 16)]
          hstore(negs_v, eidx * _NEG + k, acc)
        im = [img_v[e, pl.ds(16 * j, 16)] for j in range(4)]
        sm = [s4_v[e, pl.ds(16 * j, 16)] for j in range(4)]
        acc_di = a[0] * im[0]
        acc_ds = a[0] * sm[0]
        acc_na = a[0] * a[0]
        acc_ni = im[0] * im[0]
        acc_ns = sm[0] * sm[0]
        for j in range(1, 4):
          acc_di += a[j] * im[j]
          acc_ds += a[j] * sm[j]
          acc_na += a[j] * a[j]
          acc_ni += im[j] * im[j]
          acc_ns += sm[j] * sm[j]
        hstore(dimg_v, eidx, acc_di)
        hstore(ds4_v, eidx, acc_ds)
        hstore(nin_v, eidx, acc_na)
        hstore(nimg_v, eidx, acc_ni)
        hstore(ns4_v, eidx, acc_ns)
        return carry

      lax.fori_loop(0, _CHUNK, dot_body, 0)

      for t in range(_CHUNK // 16):
        sl = pl.ds(16 * t, 16)
        denp_v[sl] = nin_v[sl] * nimg_v[sl]
        denn_v[sl] = nin_v[sl] * ns4_v[sl]

      pltpu.sync_copy(pos_v, pos_hbm.at[pl.ds(gbase, _CHUNK)])
      pltpu.sync_copy(negs_v, negs_hbm.at[pl.ds(gbase * _NEG, _CHUNK * _NEG)])
      pltpu.sync_copy(dimg_v, dimg_hbm.at[pl.ds(gbase, _CHUNK)])
      pltpu.sync_copy(ds4_v, ds4_hbm.at[pl.ds(gbase, _CHUNK)])
      pltpu.sync_copy(denp_v, denp_hbm.at[pl.ds(gbase, _CHUNK)])
      pltpu.sync_copy(denn_v, denn_hbm.at[pl.ds(gbase, _CHUNK)])

  return sc(w_in, w_ctx, iw, cw, neg_t, img, s4)


def _tc_epilogue(pos, negs, dimg, ds4, denp, denn):
  def body(pos_ref, negs_ref, dimg_ref, ds4_ref, denp_ref, denn_ref, out_ref):
    def ls(z):
      return jnp.minimum(z, 0.0) - jnp.log1p(jnp.exp(-jnp.abs(z)))

    t1 = jnp.sum(ls(pos_ref[...])) + jnp.sum(ls(-negs_ref[...]))
    cp = dimg_ref[...] / jnp.maximum(jnp.sqrt(denp_ref[...]), 1e-8)
    cn = ds4_ref[...] / jnp.maximum(jnp.sqrt(denn_ref[...]), 1e-8)
    t2 = jnp.sum(cp - cn)
    out_ref[...] = jnp.reshape(-(t1 / _B) - t2, (1, 1))

  return pl.pallas_call(
      body,
      out_shape=jax.ShapeDtypeStruct((1, 1), jnp.float32),
  )(pos.reshape(128, 128), negs.reshape(_B * _NEG // 128, 128),
    dimg.reshape(128, 128), ds4.reshape(128, 128),
    denp.reshape(128, 128), denn.reshape(128, 128))


def kernel(input_word, context_word, img, samples, W_in, W_ctx):
  iw = input_word.astype(jnp.int32)
  cw = context_word.astype(jnp.int32)
  neg = jax.random.randint(jax.random.key(42), (_B, _NEG), 0, _V)
  neg_t = neg.T.astype(jnp.int32)
  s4 = samples[samples.shape[0] - 1]
  pos, negs, dimg, ds4, denp, denn = _sc_gather_dots(
      W_in, W_ctx, iw, cw, neg_t, img, s4)
  out = _tc_epilogue(pos, negs, dimg, ds4, denp, denn)
  return out.reshape(())
